# baseline (speedup 1.0000x reference)
import jax
import jax.numpy as jnp
from jax.experimental import pallas as pl
from jax.experimental.pallas import tpu as pltpu


def _nb_mean(x, jshift=1):
    return (jnp.roll(x, 1, 0) + jnp.roll(x, -1, 0)
            + jnp.roll(x, jshift, 1) + jnp.roll(x, -jshift, 1)) * 0.25


def _sage(x, ws, wn, b, jshift=1):
    n = x.shape[0] * x.shape[1]
    m = _nb_mean(x, jshift)
    y = x.reshape(n, -1) @ ws + m.reshape(n, -1) @ wn + b
    return jax.nn.relu(y).reshape(x.shape[0], x.shape[1], -1)


def _pool_rep(x):
    nx2, ny2, c = x.shape
    a = x.reshape(nx2 // 2, 2, ny2, c)
    a = a[:, 0] + a[:, 1]
    s = a + jnp.roll(a, -1, axis=1)
    evenj = (jax.lax.broadcasted_iota(jnp.int32, (1, ny2, 1), 1) % 2) == 0
    return 0.25 * jnp.where(evenj, s, jnp.roll(s, 1, axis=1))


def _upsample_rep(x):
    nx, ny, c = x.shape
    return jnp.broadcast_to(x[:, None], (nx, 2, ny, c)).reshape(2 * nx, ny, c)


def _unet_pair(x0_ref, x1_ref,
               dc1a0, dc1a1, dc1n0, dc1n1, dc1b,
               dc2ws, dc2wn, dc2b,
               lc1ws, lc1wn, lc1b, lc2ws, lc2wn, lc2b,
               uc1ws_u, uc1ws_s, uc1wn_u, uc1wn_s, uc1b,
               uc2ws, uc2wn, uc2b,
               out_ref):
    h = out_ref.shape[-1]
    nx, ny, cin = x0_ref.shape[2:]
    f0 = x0_ref[0, 0].reshape(nx * ny, cin)
    f1 = x1_ref[0, 0].reshape(nx * ny, cin)
    zs = f0 @ dc1a0[...] + f1 @ dc1a1[...]
    zn = (f0 @ dc1n0[...] + f1 @ dc1n1[...]).reshape(nx, ny, -1)
    x = jax.nn.relu(zs.reshape(nx, ny, -1) + _nb_mean(zn) + dc1b[...])
    x = _sage(x, dc2ws[...], dc2wn[...], dc2b[...])
    skip = x
    p = _pool_rep(x)
    p = _sage(p, lc1ws[...], lc1wn[...], lc1b[...], jshift=2)
    p = _sage(p, lc2ws[...], lc2wn[...], lc2b[...], jshift=2)
    u = _upsample_rep(p)
    uf = u.reshape(nx * ny, -1)
    sf = skip.reshape(nx * ny, -1)
    hs = uf @ uc1ws_u[...] + sf @ uc1ws_s[...]
    hn = (uf @ uc1wn_u[...] + sf @ uc1wn_s[...]).reshape(nx, ny, -1)
    y = jax.nn.relu(hs.reshape(nx, ny, -1) + _nb_mean(hn) + uc1b[...])
    y = _sage(y, uc2ws[...], uc2wn[...], uc2b[...])
    out_ref[0, 0] = y[..., :h]
    out_ref[0, 1] = y[..., h:]


def _diag2(w):
    ci, co = w.shape
    z = jnp.zeros_like(w)
    return jnp.concatenate(
        [jnp.concatenate([w, z], axis=1), jnp.concatenate([z, w], axis=1)],
        axis=0)


def _halves(w):
    z = jnp.zeros_like(w)
    return jnp.concatenate([w, z], axis=1), jnp.concatenate([z, w], axis=1)


def kernel(inputs,
           down_c1_ws, down_c1_wn, down_c1_b,
           down_c2_ws, down_c2_wn, down_c2_b,
           low_c1_ws, low_c1_wn, low_c1_b,
           low_c2_ws, low_c2_wn, low_c2_b,
           up_c1_ws, up_c1_wn, up_c1_b,
           up_c2_ws, up_c2_wn, up_c2_b):
    B, T, NX, NY, CIN = inputs.shape
    H = down_c1_ws.shape[1]
    G = (B * T) // 2
    TP = T // 2

    def b2(b):
        return jnp.concatenate([b, b]).reshape(1, 2 * b.shape[0])

    dc1a0, dc1a1 = _halves(down_c1_ws)
    dc1n0, dc1n1 = _halves(down_c1_wn)
    weights = (
        dc1a0, dc1a1, dc1n0, dc1n1, b2(down_c1_b),
        _diag2(down_c2_ws), _diag2(down_c2_wn), b2(down_c2_b),
        _diag2(low_c1_ws), _diag2(low_c1_wn), b2(low_c1_b),
        _diag2(low_c2_ws), _diag2(low_c2_wn), b2(low_c2_b),
        _diag2(up_c1_ws[:H]), _diag2(up_c1_ws[H:]),
        _diag2(up_c1_wn[:H]), _diag2(up_c1_wn[H:]), b2(up_c1_b),
        _diag2(up_c2_ws), _diag2(up_c2_wn), b2(up_c2_b),
    )

    in_specs = [
        pl.BlockSpec((1, 1, NX, NY, CIN),
                     lambda i: (i // TP, 2 * (i % TP), 0, 0, 0)),
        pl.BlockSpec((1, 1, NX, NY, CIN),
                     lambda i: (i // TP, 2 * (i % TP) + 1, 0, 0, 0)),
    ]
    for w in weights:
        in_specs.append(pl.BlockSpec(w.shape, lambda i: (0,) * w.ndim))

    out = pl.pallas_call(
        _unet_pair,
        grid=(G,),
        in_specs=in_specs,
        out_specs=pl.BlockSpec((1, 2, NX, NY, H),
                               lambda i: (i // TP, i % TP, 0, 0, 0)),
        out_shape=jax.ShapeDtypeStruct((B, T, NX, NY, H), jnp.float32),
        compiler_params=pltpu.CompilerParams(
            vmem_limit_bytes=64 * 1024 * 1024),
    )(inputs, inputs, *weights)
    return out


# speedup vs baseline: 1.1410x; 1.1410x over previous
import jax
import jax.numpy as jnp
from jax.experimental import pallas as pl
from jax.experimental.pallas import tpu as pltpu


def _nb_mean(x, jshift=1):
    return (jnp.roll(x, 1, 0) + jnp.roll(x, -1, 0)
            + jnp.roll(x, jshift, 1) + jnp.roll(x, -jshift, 1)) * 0.25


def _sage(x, ws, wn, b, jshift=1):
    n = x.shape[0] * x.shape[1]
    m = _nb_mean(x, jshift)
    y = x.reshape(n, -1) @ ws + m.reshape(n, -1) @ wn + b
    return jax.nn.relu(y).reshape(x.shape[0], x.shape[1], -1)


def _pool_rep(x):
    nx2, ny2, c = x.shape
    a = x.reshape(nx2 // 2, 2, ny2, c)
    a = a[:, 0] + a[:, 1]
    s = a + jnp.roll(a, -1, axis=1)
    evenj = (jax.lax.broadcasted_iota(jnp.int32, (1, ny2, 1), 1) % 2) == 0
    return 0.25 * jnp.where(evenj, s, jnp.roll(s, 1, axis=1))


def _upsample_rep(x):
    nx, ny, c = x.shape
    return jnp.broadcast_to(x[:, None], (nx, 2, ny, c)).reshape(2 * nx, ny, c)


def _diag2(w):
    z = jnp.zeros_like(w)
    return jnp.concatenate(
        [jnp.concatenate([w, z], axis=1), jnp.concatenate([z, w], axis=1)],
        axis=0)


def _b2(bref):
    b = bref[...]
    return jnp.concatenate([b, b], axis=1)


def _unet_pair(x_ref,
               dc1ws, dc1wn, dc1b, dc2ws, dc2wn, dc2b,
               lc1ws, lc1wn, lc1b, lc2ws, lc2wn, lc2b,
               uc1ws, uc1wn, uc1b, uc2ws, uc2wn, uc2b,
               out_ref,
               dc1d, dc2d, lc1d, lc2d,
               uc1du, uc1ds, uc1nu, uc1ns, uc2d):
    h = out_ref.shape[-1]

    @pl.when(pl.program_id(0) == 0)
    def _build_weights():
        dc1d[...] = jnp.concatenate(
            [_diag2(dc1ws[...]), _diag2(dc1wn[...])], axis=0)
        dc2d[...] = jnp.concatenate(
            [_diag2(dc2ws[...]), _diag2(dc2wn[...])], axis=0)
        lc1d[...] = jnp.concatenate(
            [_diag2(lc1ws[...]), _diag2(lc1wn[...])], axis=0)
        lc2d[...] = jnp.concatenate(
            [_diag2(lc2ws[...]), _diag2(lc2wn[...])], axis=0)
        uc1du[...] = _diag2(uc1ws[0:h])
        uc1ds[...] = _diag2(uc1ws[h:])
        uc1nu[...] = _diag2(uc1wn[0:h])
        uc1ns[...] = _diag2(uc1wn[h:])
        uc2d[...] = jnp.concatenate(
            [_diag2(uc2ws[...]), _diag2(uc2wn[...])], axis=0)

    cin2 = 2 * x_ref.shape[-1]
    x = jnp.concatenate([x_ref[0, 0], x_ref[0, 1]], axis=-1)
    x = _sage(x, dc1d[:cin2], dc1d[cin2:], _b2(dc1b))
    x = _sage(x, dc2d[:2 * h], dc2d[2 * h:], _b2(dc2b))
    skip = x
    p = _pool_rep(x)
    p = _sage(p, lc1d[:2 * h], lc1d[2 * h:], _b2(lc1b), jshift=2)
    p = _sage(p, lc2d[:2 * h], lc2d[2 * h:], _b2(lc2b), jshift=2)
    u = _upsample_rep(p)
    nx, ny = u.shape[0], u.shape[1]
    uf = u.reshape(nx * ny, -1)
    sf = skip.reshape(nx * ny, -1)
    hs = uf @ uc1du[...] + sf @ uc1ds[...]
    hn = (uf @ uc1nu[...] + sf @ uc1ns[...]).reshape(nx, ny, -1)
    y = jax.nn.relu(hs.reshape(nx, ny, -1) + _nb_mean(hn) + _b2(uc1b))
    y = _sage(y, uc2d[:2 * h], uc2d[2 * h:], _b2(uc2b))
    out_ref[0, 0] = y[..., :h]
    out_ref[0, 1] = y[..., h:]


def kernel(inputs,
           down_c1_ws, down_c1_wn, down_c1_b,
           down_c2_ws, down_c2_wn, down_c2_b,
           low_c1_ws, low_c1_wn, low_c1_b,
           low_c2_ws, low_c2_wn, low_c2_b,
           up_c1_ws, up_c1_wn, up_c1_b,
           up_c2_ws, up_c2_wn, up_c2_b):
    B, T, NX, NY, CIN = inputs.shape
    H = down_c1_ws.shape[1]
    G = (B * T) // 2
    TP = T // 2

    raw = (down_c1_ws, down_c1_wn, down_c1_b.reshape(1, H),
           down_c2_ws, down_c2_wn, down_c2_b.reshape(1, H),
           low_c1_ws, low_c1_wn, low_c1_b.reshape(1, H),
           low_c2_ws, low_c2_wn, low_c2_b.reshape(1, H),
           up_c1_ws, up_c1_wn, up_c1_b.reshape(1, H),
           up_c2_ws, up_c2_wn, up_c2_b.reshape(1, H))

    in_specs = [pl.BlockSpec((1, 2, NX, NY, CIN),
                             lambda i: (i // TP, i % TP, 0, 0, 0))]
    for w in raw:
        in_specs.append(pl.BlockSpec(w.shape, lambda i: (0,) * w.ndim))

    f32 = jnp.float32
    scratch_shapes = [
        pltpu.VMEM((2 * 2 * CIN, 2 * H), f32),
        pltpu.VMEM((4 * H, 2 * H), f32),
        pltpu.VMEM((4 * H, 2 * H), f32),
        pltpu.VMEM((4 * H, 2 * H), f32),
        pltpu.VMEM((2 * H, 2 * H), f32),
        pltpu.VMEM((2 * H, 2 * H), f32),
        pltpu.VMEM((2 * H, 2 * H), f32),
        pltpu.VMEM((2 * H, 2 * H), f32),
        pltpu.VMEM((4 * H, 2 * H), f32),
    ]

    out = pl.pallas_call(
        _unet_pair,
        grid=(G,),
        in_specs=in_specs,
        out_specs=pl.BlockSpec((1, 2, NX, NY, H),
                               lambda i: (i // TP, i % TP, 0, 0, 0)),
        out_shape=jax.ShapeDtypeStruct((B, T, NX, NY, H), jnp.float32),
        scratch_shapes=scratch_shapes,
        compiler_params=pltpu.CompilerParams(
            vmem_limit_bytes=64 * 1024 * 1024),
    )(inputs, *raw)
    return out
